# baseline (speedup 1.0000x reference)
import functools

import jax
import jax.numpy as jnp
from jax.experimental import pallas as pl
from jax.experimental.pallas import tpu as pltpu

LANE = 128
_VMEM_LIMIT = 44 * 1024 * 1024


def _round_up(x, m):
    return (x + m - 1) // m * m


def _pick_tile(dim_p, pref):
    t = max(LANE, min(pref, dim_p))
    t = (t // LANE) * LANE
    while dim_p % t:
        t -= LANE
    return t


def _make_body(ni, num_b, n_mchunks, m_chunk, has_tail):
    xw_step = ni if has_tail else ni - 1

    def _gcn_body(alpha_ref, seq_ref, adj_ref, w_ref, bias_ref, o_ref, xw_ref):
        b = pl.program_id(0)
        i = pl.program_id(1)
        slot = jax.lax.rem(b, 2)

        def _xw(dst_slot):
            xw_ref[dst_slot] = jnp.dot(
                seq_ref[...], w_ref[...],
                preferred_element_type=jnp.float32)

        @pl.when(jnp.logical_and(b == 0, i == 0))
        def _():
            _xw(0)

        @pl.when(i < ni)
        def _():
            alpha = alpha_ref[0]
            for m in range(n_mchunks):
                sl = pl.ds(m * m_chunk, m_chunk)
                h = jnp.dot(adj_ref[sl, :], xw_ref[slot],
                            preferred_element_type=jnp.float32) + bias_ref[...]
                o_ref[sl, :] = jnp.where(h > 0.0, h, alpha * h)

        @pl.when(jnp.logical_and(i == xw_step, b < num_b - 1))
        def _():
            _xw(1 - slot)

    return _gcn_body


@jax.jit
def kernel(seq, adj, w, bias, alpha):
    B, N, F_in = seq.shape
    F_h = w.shape[1]
    alpha1d = jnp.asarray(alpha, jnp.float32).reshape(1)

    Np = _round_up(N, LANE)
    Fi = _round_up(F_in, LANE)
    Fh = _round_up(F_h, LANE)
    seq_p = seq.astype(jnp.float32)
    if Np != N or Fi != F_in:
        seq_p = jnp.pad(seq_p, ((0, 0), (0, Np - N), (0, Fi - F_in)))
    adj_p = adj.astype(jnp.float32)
    if Np != N:
        adj_p = jnp.pad(adj_p, ((0, 0), (0, Np - N), (0, Np - N)))
    w_p = w.astype(jnp.float32)
    if Fi != F_in or Fh != F_h:
        w_p = jnp.pad(w_p, ((0, Fi - F_in), (0, Fh - F_h)))
    bias_p = bias
    if Fh != F_h:
        bias_p = jnp.pad(bias_p, (0, Fh - F_h))
    bias_p = bias_p.reshape(1, Fh).astype(jnp.float32)

    tn = _pick_tile(Np, 1024)

    def _vmem_bytes(tn_):
        return (2 * tn_ * Np * 4
                + 2 * Np * Fi * 4
                + 2 * tn_ * Fh * 4
                + 2 * Np * Fh * 4
                + 2 * Fi * Fh * 4)

    while _vmem_bytes(tn) > _VMEM_LIMIT - 2 * 1024 * 1024 and tn > LANE:
        tn = _pick_tile(Np, tn - LANE)

    ni = Np // tn
    has_tail = True
    grid = (B, ni + 1) if has_tail else (B, ni)
    flops = 2 * B * (Np * Fi * Fh + Np * Np * Fh)
    bytes_accessed = (B * (Np * Np + Np * Fi + Np * Fh) * 4
                      + Fi * Fh * 2 + Fh * 4)

    seq_switch = ni if has_tail else ni - 1

    def _adj_idx(b, i):
        return (b, jnp.minimum(i, ni - 1), 0)

    def _out_idx(b, i):
        return (b, jnp.minimum(i, ni - 1), 0)

    def _seq_idx(b, i):
        return (jnp.minimum(jnp.where(i >= seq_switch, b + 1, b), B - 1), 0, 0)

    n_mchunks = 1
    m_chunk = tn // n_mchunks

    out = pl.pallas_call(
        _make_body(ni, B, n_mchunks, m_chunk, has_tail),
        out_shape=jax.ShapeDtypeStruct((B, Np, Fh), jnp.float32),
        grid=grid,
        in_specs=[
            pl.BlockSpec(memory_space=pltpu.MemorySpace.SMEM),
            pl.BlockSpec((None, Np, Fi), _seq_idx),
            pl.BlockSpec((None, tn, Np), _adj_idx),
            pl.BlockSpec((Fi, Fh), lambda b, i: (0, 0)),
            pl.BlockSpec((1, Fh), lambda b, i: (0, 0)),
        ],
        out_specs=pl.BlockSpec((None, tn, Fh), _out_idx),
        scratch_shapes=[pltpu.VMEM((2, Np, Fh), jnp.float32)],
        compiler_params=pltpu.CompilerParams(
            dimension_semantics=("arbitrary", "arbitrary"),
            vmem_limit_bytes=_VMEM_LIMIT),
        cost_estimate=pl.CostEstimate(flops=flops, transcendentals=0,
                                      bytes_accessed=bytes_accessed),
    )(alpha1d, seq_p, adj_p, w_p, bias_p)
    if Np != N or Fh != F_h:
        out = out[:, :N, :F_h]
    return out


# speedup vs baseline: 1.1681x; 1.1681x over previous
import jax
import jax.numpy as jnp
from jax.experimental import pallas as pl
from jax.experimental.pallas import tpu as pltpu

LANE = 128
_VMEM_LIMIT = 48 * 1024 * 1024


def _round_up(x, m):
    return (x + m - 1) // m * m


def _pick_tile(dim_p, pref):
    t = max(LANE, min(pref, dim_p))
    t = (t // LANE) * LANE
    while dim_p % t:
        t -= LANE
    return t


def _make_body(B, Np, Fi, Fh, tn):
    ni = Np // tn
    total = B * ni

    def body(alpha_ref, seq_hbm, adj_hbm, w_hbm, bias_hbm, out_hbm,
             adj_buf, seq_buf, xw_buf, out_buf, w_buf, bias_buf,
             adj_sem, seq_sem, out_sem, w_sem, bias_sem):
        def adj_copy(s):
            b, i = divmod(s, ni)
            return pltpu.make_async_copy(
                adj_hbm.at[b, pl.ds(i * tn, tn), :], adj_buf.at[s % 3],
                adj_sem.at[s % 3])

        def seq_copy(b):
            return pltpu.make_async_copy(
                seq_hbm.at[b], seq_buf.at[b % 2], seq_sem.at[b % 2])

        def out_copy(s):
            b, i = divmod(s, ni)
            return pltpu.make_async_copy(
                out_buf.at[s % 2], out_hbm.at[b, pl.ds(i * tn, tn), :],
                out_sem.at[s % 2])

        w_c = pltpu.make_async_copy(w_hbm, w_buf, w_sem)
        bias_c = pltpu.make_async_copy(bias_hbm, bias_buf, bias_sem)

        seq_copy(0).start()
        w_c.start()
        bias_c.start()
        adj_copy(0).start()
        if total > 1:
            adj_copy(1).start()

        seq_copy(0).wait()
        w_c.wait()
        xw_buf[0] = jnp.dot(seq_buf[0], w_buf[...],
                            preferred_element_type=jnp.float32)
        bias_c.wait()

        alpha = alpha_ref[0]
        out_started = []
        for s in range(total):
            b, i = divmod(s, ni)
            if i == 0 and b + 1 < B:
                seq_copy(b + 1).start()
            if s + 2 < total:
                adj_copy(s + 2).start()
            adj_copy(s).wait()
            if len(out_started) >= 2:
                out_copy(out_started.pop(0)).wait()
            h = jnp.dot(adj_buf[s % 3], xw_buf[b % 2],
                        preferred_element_type=jnp.float32) + bias_buf[...]
            out_buf[s % 2] = jnp.where(h > 0.0, h, alpha * h)
            out_copy(s).start()
            out_started.append(s)
            if i == ni - 1 and b + 1 < B:
                seq_copy(b + 1).wait()
                xw_buf[(b + 1) % 2] = jnp.dot(
                    seq_buf[(b + 1) % 2], w_buf[...],
                    preferred_element_type=jnp.float32)
        for s in out_started:
            out_copy(s).wait()

    return body


@jax.jit
def kernel(seq, adj, w, bias, alpha):
    B, N, F_in = seq.shape
    F_h = w.shape[1]
    alpha1d = jnp.asarray(alpha, jnp.float32).reshape(1)

    Np = _round_up(N, LANE)
    Fi = _round_up(F_in, LANE)
    Fh = _round_up(F_h, LANE)
    seq_p = seq.astype(jnp.float32)
    if Np != N or Fi != F_in:
        seq_p = jnp.pad(seq_p, ((0, 0), (0, Np - N), (0, Fi - F_in)))
    adj_p = adj.astype(jnp.float32)
    if Np != N:
        adj_p = jnp.pad(adj_p, ((0, 0), (0, Np - N), (0, Np - N)))
    w_p = w.astype(jnp.float32)
    if Fi != F_in or Fh != F_h:
        w_p = jnp.pad(w_p, ((0, Fi - F_in), (0, Fh - F_h)))
    bias_p = bias
    if Fh != F_h:
        bias_p = jnp.pad(bias_p, (0, Fh - F_h))
    bias_p = bias_p.reshape(1, Fh).astype(jnp.float32)

    tn = _pick_tile(Np, 1024)

    def _vmem_bytes(tn_):
        return (3 * tn_ * Np * 4
                + 2 * Np * Fi * 4
                + 2 * tn_ * Fh * 4
                + 2 * Np * Fh * 4
                + Fi * Fh * 4)

    while _vmem_bytes(tn) > _VMEM_LIMIT - 2 * 1024 * 1024 and tn > LANE:
        tn = _pick_tile(Np, tn - LANE)

    out = pl.pallas_call(
        _make_body(B, Np, Fi, Fh, tn),
        out_shape=jax.ShapeDtypeStruct((B, Np, Fh), jnp.float32),
        in_specs=[
            pl.BlockSpec(memory_space=pltpu.MemorySpace.SMEM),
            pl.BlockSpec(memory_space=pltpu.MemorySpace.HBM),
            pl.BlockSpec(memory_space=pltpu.MemorySpace.HBM),
            pl.BlockSpec(memory_space=pltpu.MemorySpace.HBM),
            pl.BlockSpec(memory_space=pltpu.MemorySpace.HBM),
        ],
        out_specs=pl.BlockSpec(memory_space=pltpu.MemorySpace.HBM),
        scratch_shapes=[
            pltpu.VMEM((3, tn, Np), jnp.float32),
            pltpu.VMEM((2, Np, Fi), jnp.float32),
            pltpu.VMEM((2, Np, Fh), jnp.float32),
            pltpu.VMEM((2, tn, Fh), jnp.float32),
            pltpu.VMEM((Fi, Fh), jnp.float32),
            pltpu.VMEM((1, Fh), jnp.float32),
            pltpu.SemaphoreType.DMA((3,)),
            pltpu.SemaphoreType.DMA((2,)),
            pltpu.SemaphoreType.DMA((2,)),
            pltpu.SemaphoreType.DMA,
            pltpu.SemaphoreType.DMA,
        ],
        compiler_params=pltpu.CompilerParams(
            vmem_limit_bytes=_VMEM_LIMIT),
    )(alpha1d, seq_p, adj_p, w_p, bias_p)
    if Np != N or Fh != F_h:
        out = out[:, :N, :F_h]
    return out
